# Initial kernel scaffold; baseline (speedup 1.0000x reference)
import jax
import jax.numpy as jnp
from jax.experimental import pallas as pl


def kernel(x3, x4, x5, text_feat, params):
    raise NotImplementedError("write your pallas kernel here")


# speedup vs baseline: 2.5060x; 2.5060x over previous
import functools
import math

import jax
import jax.numpy as jnp
from jax import lax
from jax.experimental import pallas as pl
from jax.experimental.pallas import tpu as pltpu

_NC = 80
_C_TEXT = 512
_EMBED = 128
_GCN_H = 64
_THR = 0.5
_LEVELS = [(192, 64), (384, 32), (768, 16)]
_K_RATIO = 0.005
_INTERPRET = False

_f32 = jnp.float32


def _silu(x):
    return x * jax.nn.sigmoid(x)


def _conv3(xin, xp_ref, w_ref, W, HW):
    cin = xin.shape[1]
    xp_ref[0:W + 1, :] = jnp.zeros((W + 1, cin), _f32)
    xp_ref[W + 1:W + 1 + HW, :] = xin
    xp_ref[W + 1 + HW:, :] = jnp.zeros((W + 1, cin), _f32)
    col = lax.broadcasted_iota(jnp.int32, (HW, 1), 0) & (W - 1)
    cout = w_ref.shape[3]
    acc = jnp.zeros((HW, cout), _f32)
    for dy in range(3):
        for dx in range(3):
            off = dy * W + dx
            tap = xp_ref[off:off + HW, :]
            if dx == 0:
                tap = jnp.where(col != 0, tap, 0.0)
            elif dx == 2:
                tap = jnp.where(col != W - 1, tap, 0.0)
            acc = acc + jax.lax.dot(tap, w_ref[dy, dx],
                                    preferred_element_type=_f32)
    return acc


def _trunk_body(H, W, ch, hid, c, ns,
                p1_ref, p2_ref, text_ref,
                wcv1_ref, bcv1_ref, wcv2_ref, bcv2_ref,
                wimg_ref, bimg_ref, twt_ref, tb_ref, ab_ref,
                wproj_ref, bproj_ref, wf_ref, bf_ref,
                fused_ref, sel_ref, top_ref,
                xp1_ref, xp2_ref):
    HW = H * W
    p1 = p1_ref[0]
    p2 = p2_ref[0]
    hidv = _silu(jax.lax.dot(p2, wcv1_ref[...], preferred_element_type=_f32)
                 + bcv1_ref[...])
    bout = p2 + _silu(_conv3(hidv, xp1_ref, wcv2_ref, W, HW) + bcv2_ref[...])
    ie = jax.lax.dot(bout, wimg_ref[...], preferred_element_type=_f32) \
        + bimg_ref[...]
    t = jax.lax.dot(text_ref[0], twt_ref[...], preferred_element_type=_f32) \
        + tb_ref[...]
    aw = lax.dot_general(ie, t, (((1,), (1,)), ((), ())),
                         preferred_element_type=_f32)
    s = jax.nn.sigmoid(jnp.max(aw, axis=1, keepdims=True)
                       * (1.0 / math.sqrt(_EMBED)) + ab_ref[...])
    attn = (_conv3(bout, xp2_ref, wproj_ref, W, HW) + bproj_ref[...]) * s
    fused = (jax.lax.dot(p1, wf_ref[0:ch, :], preferred_element_type=_f32)
             + jax.lax.dot(p2, wf_ref[ch:2 * ch, :],
                           preferred_element_type=_f32)
             + jax.lax.dot(bout, wf_ref[2 * ch:3 * ch, :],
                           preferred_element_type=_f32)
             + jax.lax.dot(attn, wf_ref[3 * ch:4 * ch, :],
                           preferred_element_type=_f32)
             + bf_ref[...])
    fused_ref[0] = fused
    row = lax.broadcasted_iota(jnp.int32, (HW, 1), 0)
    lane = lax.broadcasted_iota(jnp.int32, (1, ns), 1)
    sc = s
    sel = jnp.zeros((HW, ns), _f32)
    for j in range(ns):
        m = jnp.max(sc, axis=0, keepdims=True)
        cand = jnp.where(sc == m, row, HW)
        ix = jnp.min(cand, axis=0, keepdims=True)
        hit = (row == ix)
        sel = sel + hit.astype(_f32) * (lane == j).astype(_f32)
        sc = jnp.where(hit, -jnp.inf, sc)
    sel_ref[0] = sel
    top_ref[0] = lax.dot_general(sel, fused, (((0,), (0,)), ((), ())),
                                 preferred_element_type=_f32)


def _gcn_body(n, top_ref, w1_ref, b1_ref, w2_ref, b2_ref, u_ref):
    top = top_ref[...]
    nrm = jnp.maximum(jnp.sqrt(jnp.sum(top * top, axis=1, keepdims=True)),
                      1e-12)
    nf = top / nrm
    sim = lax.dot_general(nf, nf, (((1,), (1,)), ((), ())),
                          preferred_element_type=_f32)
    ri = lax.broadcasted_iota(jnp.int32, (n, n), 0)
    ci = lax.broadcasted_iota(jnp.int32, (n, n), 1)
    at = (sim > _THR).astype(_f32) + (ri == ci).astype(_f32)
    degr = jnp.sum(at, axis=1, keepdims=True)
    degc = jnp.sum(at, axis=0, keepdims=True)
    dr = jnp.where(degr > 0, 1.0 / jnp.sqrt(degr), 0.0)
    dc = jnp.where(degc > 0, 1.0 / jnp.sqrt(degc), 0.0)
    p = at * dr * dc
    h1 = jax.nn.relu(
        jax.lax.dot(p, jax.lax.dot(top, w1_ref[...],
                                   preferred_element_type=_f32),
                    preferred_element_type=_f32) + b1_ref[...])
    u = jax.lax.dot(p, jax.lax.dot(h1, w2_ref[...],
                                   preferred_element_type=_f32),
                    preferred_element_type=_f32) + b2_ref[...]
    u_ref[...] = u


def _scatter_body(fused_ref, sel_ref, u_ref, out_ref):
    sel = sel_ref[0]
    mask = jnp.sum(sel, axis=1, keepdims=True)
    out_ref[0] = fused_ref[0] * (1.0 - mask) + \
        jax.lax.dot(sel, u_ref[0], preferred_element_type=_f32)


def _batch_spec(shape):
    return pl.BlockSpec((1,) + tuple(shape),
                        lambda b: (b,) + (0,) * len(shape))


def _full_spec(a):
    nd = a.ndim
    return pl.BlockSpec(a.shape, lambda b, _n=nd: (0,) * _n)


def _fold1x1(p):
    w = (p['w'][:, :, 0, 0] * p['gamma'][:, None]).T
    return w, p['beta'][None, :]


def _fold3x3(p):
    w = (p['w'] * p['gamma'][:, None, None, None]).transpose(2, 3, 1, 0)
    return w, p['beta'][None, :]


def _level(xr, text_feat, p, c, hw_side, ns):
    bsz = xr.shape[0]
    H = W = hw_side
    HW = H * W
    ch = c // 2
    hid = ch // 2
    n = bsz * ns

    p1 = xr[:, :, :ch]
    p2 = xr[:, :, ch:]
    wcv1, bcv1 = _fold1x1(p['cv1'])
    wcv2, bcv2 = _fold3x3(p['cv2'])
    wimg, bimg = _fold1x1(p['img_conv'])
    twt = p['text_w'].T
    tb = p['text_b'][None, :]
    ab = p['attn_bias'].reshape(1, 1)
    wproj, bproj = _fold3x3(p['proj_conv'])
    wf, bf = _fold1x1(p['fusion'])

    trunk_in = (p1, p2, text_feat, wcv1, bcv1, wcv2, bcv2, wimg, bimg,
                twt, tb, ab, wproj, bproj, wf, bf)
    in_specs = [_batch_spec((HW, ch)), _batch_spec((HW, ch)),
                _batch_spec((_NC, _C_TEXT))] + \
        [_full_spec(a) for a in trunk_in[3:]]
    fused, sel, top = pl.pallas_call(
        functools.partial(_trunk_body, H, W, ch, hid, c, ns),
        grid=(bsz,),
        in_specs=in_specs,
        out_specs=[_batch_spec((HW, c)), _batch_spec((HW, ns)),
                   _batch_spec((ns, c))],
        out_shape=[jax.ShapeDtypeStruct((bsz, HW, c), _f32),
                   jax.ShapeDtypeStruct((bsz, HW, ns), _f32),
                   jax.ShapeDtypeStruct((bsz, ns, c), _f32)],
        scratch_shapes=[pltpu.VMEM((HW + 2 * W + 2, hid), _f32),
                        pltpu.VMEM((HW + 2 * W + 2, ch), _f32)],
        interpret=_INTERPRET,
    )(*trunk_in)

    topf = top.reshape(n, c)
    gcn_in = (topf, p['gcn1_w'], p['gcn1_b'][None, :],
              p['gcn2_w'], p['gcn2_b'][None, :])
    u = pl.pallas_call(
        functools.partial(_gcn_body, n),
        in_specs=[pl.BlockSpec(a.shape, functools.partial(
            lambda _n: (0,) * _n, a.ndim)) for a in gcn_in],
        out_specs=pl.BlockSpec((n, c), lambda: (0, 0)),
        out_shape=jax.ShapeDtypeStruct((n, c), _f32),
        interpret=_INTERPRET,
    )(*gcn_in)

    out = pl.pallas_call(
        _scatter_body,
        grid=(bsz,),
        in_specs=[_batch_spec((HW, c)), _batch_spec((HW, ns)),
                  _batch_spec((ns, c))],
        out_specs=_batch_spec((HW, c)),
        out_shape=jax.ShapeDtypeStruct((bsz, HW, c), _f32),
        interpret=_INTERPRET,
    )(fused, sel, u.reshape(bsz, ns, c))
    return out.transpose(0, 2, 1).reshape(bsz, c, H, W)


def kernel(x3, x4, x5, text_feat, params):
    outs = []
    for i, (x, (c, hw)) in enumerate(zip((x3, x4, x5), _LEVELS)):
        ns = int(hw * hw * _K_RATIO)
        bsz = x.shape[0]
        xr = x.reshape(bsz, c, hw * hw).transpose(0, 2, 1)
        outs.append(_level(xr, text_feat, params['l%d' % i], c, hw, ns))
    return tuple(outs)


# speedup vs baseline: 3.3887x; 1.3522x over previous
import functools
import math

import jax
import jax.numpy as jnp
from jax import lax
from jax.experimental import pallas as pl
from jax.experimental.pallas import tpu as pltpu

_NC = 80
_C_TEXT = 512
_EMBED = 128
_GCN_H = 64
_THR = 0.5
_LEVELS = [(192, 64), (384, 32), (768, 16)]
_K_RATIO = 0.005
_INTERPRET = False

_f32 = jnp.float32


def _silu(x):
    return x * jax.nn.sigmoid(x)


def _conv3(xin, xp_ref, w_ref, W, HW):
    cin = xin.shape[1]
    xp_ref[0:W + 1, :] = jnp.zeros((W + 1, cin), _f32)
    xp_ref[W + 1:W + 1 + HW, :] = xin
    xp_ref[W + 1 + HW:, :] = jnp.zeros((W + 1, cin), _f32)
    col = lax.broadcasted_iota(jnp.int32, (HW, 1), 0) & (W - 1)
    cout = w_ref.shape[3]
    acc = jnp.zeros((HW, cout), _f32)
    for dy in range(3):
        for dx in range(3):
            off = dy * W + dx
            tap = xp_ref[off:off + HW, :]
            if dx == 0:
                tap = jnp.where(col != 0, tap, 0.0)
            elif dx == 2:
                tap = jnp.where(col != W - 1, tap, 0.0)
            acc = acc + jax.lax.dot(tap, w_ref[dy, dx],
                                    preferred_element_type=_f32)
    return acc


def _trunk_body(H, W, ch, hid, c, ns,
                p1_ref, p2_ref, text_ref,
                wcv1_ref, bcv1_ref, wcv2_ref, bcv2_ref,
                wimg_ref, bimg_ref, twt_ref, tb_ref, ab_ref,
                wproj_ref, bproj_ref, wf_ref, bf_ref,
                fused_ref, sel_ref, top_ref,
                xp1_ref, xp2_ref):
    HW = H * W
    p1 = p1_ref[0]
    p2 = p2_ref[0]
    hidv = _silu(jax.lax.dot(p2, wcv1_ref[...], preferred_element_type=_f32)
                 + bcv1_ref[...])
    bout = p2 + _silu(_conv3(hidv, xp1_ref, wcv2_ref, W, HW) + bcv2_ref[...])
    ie = jax.lax.dot(bout, wimg_ref[...], preferred_element_type=_f32) \
        + bimg_ref[...]
    t = jax.lax.dot(text_ref[0], twt_ref[...], preferred_element_type=_f32) \
        + tb_ref[...]
    aw = lax.dot_general(ie, t, (((1,), (1,)), ((), ())),
                         preferred_element_type=_f32)
    s = jax.nn.sigmoid(jnp.max(aw, axis=1, keepdims=True)
                       * (1.0 / math.sqrt(_EMBED)) + ab_ref[...])
    awt = lax.dot_general(t, ie, (((1,), (1,)), ((), ())),
                          preferred_element_type=_f32)
    s_row = jax.nn.sigmoid(jnp.max(awt, axis=0, keepdims=True)
                           * (1.0 / math.sqrt(_EMBED)) + ab_ref[...])
    attn = (_conv3(bout, xp2_ref, wproj_ref, W, HW) + bproj_ref[...]) * s
    fused = (jax.lax.dot(p1, wf_ref[0:ch, :], preferred_element_type=_f32)
             + jax.lax.dot(p2, wf_ref[ch:2 * ch, :],
                           preferred_element_type=_f32)
             + jax.lax.dot(bout, wf_ref[2 * ch:3 * ch, :],
                           preferred_element_type=_f32)
             + jax.lax.dot(attn, wf_ref[3 * ch:4 * ch, :],
                           preferred_element_type=_f32)
             + bf_ref[...])
    fused_ref[0] = fused
    lane = lax.broadcasted_iota(jnp.int32, (1, HW), 1)
    sub = lax.broadcasted_iota(jnp.int32, (ns, 1), 0)
    sc = s_row
    r = jnp.zeros((1, HW), jnp.int32)
    for j in range(ns):
        m = jnp.max(sc, axis=1, keepdims=True)
        cand = jnp.where(sc == m, lane, HW)
        ix = jnp.min(cand, axis=1, keepdims=True)
        hit = (lane == ix)
        r = r + hit.astype(jnp.int32) * (j + 1)
        sc = jnp.where(hit, -jnp.inf, sc)
    sel = (r == sub + 1).astype(_f32)
    sel_ref[0] = sel
    top_ref[0] = jax.lax.dot(sel, fused, preferred_element_type=_f32)


def _gcn_body(n, top_ref, w1_ref, b1_ref, w2_ref, b2_ref, u_ref):
    top = top_ref[...]
    nrm = jnp.maximum(jnp.sqrt(jnp.sum(top * top, axis=1, keepdims=True)),
                      1e-12)
    nf = top / nrm
    sim = lax.dot_general(nf, nf, (((1,), (1,)), ((), ())),
                          preferred_element_type=_f32)
    ri = lax.broadcasted_iota(jnp.int32, (n, n), 0)
    ci = lax.broadcasted_iota(jnp.int32, (n, n), 1)
    at = (sim > _THR).astype(_f32) + (ri == ci).astype(_f32)
    degr = jnp.sum(at, axis=1, keepdims=True)
    degc = jnp.sum(at, axis=0, keepdims=True)
    dr = jnp.where(degr > 0, 1.0 / jnp.sqrt(degr), 0.0)
    dc = jnp.where(degc > 0, 1.0 / jnp.sqrt(degc), 0.0)
    p = at * dr * dc
    h1 = jax.nn.relu(
        jax.lax.dot(p, jax.lax.dot(top, w1_ref[...],
                                   preferred_element_type=_f32),
                    preferred_element_type=_f32) + b1_ref[...])
    u = jax.lax.dot(p, jax.lax.dot(h1, w2_ref[...],
                                   preferred_element_type=_f32),
                    preferred_element_type=_f32) + b2_ref[...]
    u_ref[...] = u


def _scatter_body(fused_ref, sel_ref, u_ref, top_ref, out_ref):
    delta = u_ref[0] - top_ref[0]
    out_ref[0] = fused_ref[0] + lax.dot_general(
        sel_ref[0], delta, (((0,), (0,)), ((), ())),
        preferred_element_type=_f32)


def _batch_spec(shape):
    return pl.BlockSpec((1,) + tuple(shape),
                        lambda b: (b,) + (0,) * len(shape))


def _full_spec(a):
    nd = a.ndim
    return pl.BlockSpec(a.shape, lambda b, _n=nd: (0,) * _n)


def _fold1x1(p):
    w = (p['w'][:, :, 0, 0] * p['gamma'][:, None]).T
    return w, p['beta'][None, :]


def _fold3x3(p):
    w = (p['w'] * p['gamma'][:, None, None, None]).transpose(2, 3, 1, 0)
    return w, p['beta'][None, :]


def _level(xr, text_feat, p, c, hw_side, ns):
    bsz = xr.shape[0]
    H = W = hw_side
    HW = H * W
    ch = c // 2
    hid = ch // 2
    n = bsz * ns

    p1 = xr[:, :, :ch]
    p2 = xr[:, :, ch:]
    wcv1, bcv1 = _fold1x1(p['cv1'])
    wcv2, bcv2 = _fold3x3(p['cv2'])
    wimg, bimg = _fold1x1(p['img_conv'])
    twt = p['text_w'].T
    tb = p['text_b'][None, :]
    ab = p['attn_bias'].reshape(1, 1)
    wproj, bproj = _fold3x3(p['proj_conv'])
    wf, bf = _fold1x1(p['fusion'])

    trunk_in = (p1, p2, text_feat, wcv1, bcv1, wcv2, bcv2, wimg, bimg,
                twt, tb, ab, wproj, bproj, wf, bf)
    in_specs = [_batch_spec((HW, ch)), _batch_spec((HW, ch)),
                _batch_spec((_NC, _C_TEXT))] + \
        [_full_spec(a) for a in trunk_in[3:]]
    fused, sel, top = pl.pallas_call(
        functools.partial(_trunk_body, H, W, ch, hid, c, ns),
        grid=(bsz,),
        in_specs=in_specs,
        out_specs=[_batch_spec((HW, c)), _batch_spec((ns, HW)),
                   _batch_spec((ns, c))],
        out_shape=[jax.ShapeDtypeStruct((bsz, HW, c), _f32),
                   jax.ShapeDtypeStruct((bsz, ns, HW), _f32),
                   jax.ShapeDtypeStruct((bsz, ns, c), _f32)],
        scratch_shapes=[pltpu.VMEM((HW + 2 * W + 2, hid), _f32),
                        pltpu.VMEM((HW + 2 * W + 2, ch), _f32)],
        interpret=_INTERPRET,
    )(*trunk_in)

    topf = top.reshape(n, c)
    gcn_in = (topf, p['gcn1_w'], p['gcn1_b'][None, :],
              p['gcn2_w'], p['gcn2_b'][None, :])
    u = pl.pallas_call(
        functools.partial(_gcn_body, n),
        in_specs=[pl.BlockSpec(a.shape, functools.partial(
            lambda _n: (0,) * _n, a.ndim)) for a in gcn_in],
        out_specs=pl.BlockSpec((n, c), lambda: (0, 0)),
        out_shape=jax.ShapeDtypeStruct((n, c), _f32),
        interpret=_INTERPRET,
    )(*gcn_in)

    out = pl.pallas_call(
        _scatter_body,
        grid=(bsz,),
        in_specs=[_batch_spec((HW, c)), _batch_spec((ns, HW)),
                  _batch_spec((ns, c)), _batch_spec((ns, c))],
        out_specs=_batch_spec((HW, c)),
        out_shape=jax.ShapeDtypeStruct((bsz, HW, c), _f32),
        interpret=_INTERPRET,
    )(fused, sel, u.reshape(bsz, ns, c), top)
    return out.transpose(0, 2, 1).reshape(bsz, c, H, W)


def kernel(x3, x4, x5, text_feat, params):
    outs = []
    for i, (x, (c, hw)) in enumerate(zip((x3, x4, x5), _LEVELS)):
        ns = int(hw * hw * _K_RATIO)
        bsz = x.shape[0]
        xr = x.reshape(bsz, c, hw * hw).transpose(0, 2, 1)
        outs.append(_level(xr, text_feat, params['l%d' % i], c, hw, ns))
    return tuple(outs)
